# Initial kernel scaffold; baseline (speedup 1.0000x reference)
import jax
import jax.numpy as jnp
from jax.experimental import pallas as pl


def kernel(embed_feature, ignore_tags, proposal_points, params):
    raise NotImplementedError("write your pallas kernel here")


# speedup vs baseline: 1.9215x; 1.9215x over previous
import functools

import jax
import jax.numpy as jnp
from jax import lax
from jax.experimental import pallas as pl
from jax.experimental.pallas import tpu as pltpu
from jax.experimental.pallas import tpu_sc as plsc

NODE = 20
N_INST = 256
NPTS = N_INST * NODE
C_REAL = 36
CPAD = 48
H = W = 256
NW = 32
PPW = NPTS // NW
ROWS_PW = 4 * PPW
IDX_CH = 128
N_CH = ROWS_PW // IDX_CH
TILE = 320
INV128 = 1.0 / 128.0


def _sc_body(px_hbm, py_hbm, table_hbm, out_hbm, px_v, py_v, idx_v, rows_v, sem):
    cid = lax.axis_index("c")
    sid = lax.axis_index("s")
    wid = sid * 2 + cid
    base = pl.multiple_of(wid * PPW, PPW)
    pltpu.sync_copy(px_hbm.at[pl.ds(base, PPW)], px_v)
    pltpu.sync_copy(py_hbm.at[pl.ds(base, PPW)], py_v)
    rowbase = (wid // 8) * (H * W)
    for g in range(PPW // 16):
        pxg = px_v[pl.ds(g * 16, 16)]
        pyg = py_v[pl.ds(g * 16, 16)]
        x = (pxg * INV128 - 1.0 + 1.0) * 127.5
        y = (pyg * INV128 - 1.0 + 1.0) * 127.5
        x0 = jnp.clip(x.astype(jnp.int32), 0, W - 1)
        y0 = jnp.clip(y.astype(jnp.int32), 0, H - 1)
        x1 = jnp.minimum(x0 + 1, W - 1)
        y1 = jnp.minimum(y0 + 1, H - 1)
        r0 = rowbase + y0 * W
        r1 = rowbase + y1 * W
        for k, r in enumerate((r0 + x0, r0 + x1, r1 + x0, r1 + x1)):
            off = k * PPW + g * 16
            idx_v[off // IDX_CH, pl.ds(off % IDX_CH, 16)] = r
    handles = [
        pltpu.async_copy(table_hbm.at[idx_v.at[j]],
                         rows_v.at[pl.ds(j * IDX_CH, IDX_CH)], sem)
        for j in range(N_CH)
    ]
    for h in handles:
        h.wait()
    pltpu.sync_copy(rows_v, out_hbm.at[wid])


@functools.cache
def _sc_gather_call():
    return pl.kernel(
        _sc_body,
        mesh=plsc.VectorSubcoreMesh(core_axis_name="c", subcore_axis_name="s"),
        compiler_params=pltpu.CompilerParams(use_tc_tiling_on_sc=False),
        out_type=jax.ShapeDtypeStruct((NW, ROWS_PW, CPAD), jnp.float32),
        scratch_types=[
            pltpu.VMEM((PPW,), jnp.float32),
            pltpu.VMEM((PPW,), jnp.float32),
            pltpu.VMEM((N_CH, IDX_CH), jnp.int32),
            pltpu.VMEM((ROWS_PW, CPAD), jnp.float32),
            pltpu.SemaphoreType.DMA,
        ],
    )


def _ring_avg(xx, node):
    n = xx.shape[0]
    acc = xx
    for k in (1, 2):
        a = jnp.concatenate([xx[n - k:], xx[:n - k]], axis=0)
        b = jnp.concatenate([xx[NODE - k:], xx[:NODE - k]], axis=0)
        acc = acc + jnp.where(node < k, b, a)
        c = jnp.concatenate([xx[k:], xx[:k]], axis=0)
        d = jnp.concatenate([xx[n - (NODE - k):], xx[:n - (NODE - k)]], axis=0)
        acc = acc + jnp.where(node >= NODE - k, d, c)
    return acc * 0.2


def _tc_body(rows_ref, polys_ref, w1, b1, w2, b2, w3, b3, w4, b4,
             p1, q1, p2, q2, p3, q3, out_ref, nf_scr):
    for wslot in range(NW):
        rw = rows_ref[wslot]
        pts = polys_ref[pl.ds(wslot * PPW, PPW), :]
        x = (pts[:, 0:1] * INV128 - 1.0 + 1.0) * 127.5
        y = (pts[:, 1:2] * INV128 - 1.0 + 1.0) * 127.5
        x0 = jnp.floor(x)
        y0 = jnp.floor(y)
        wx1 = x - x0
        wx0 = 1.0 - wx1
        wy1 = y - y0
        wy0 = 1.0 - wy1
        nf_scr[pl.ds(wslot * PPW, PPW), :] = (
            rw[0:PPW] * (wx0 * wy0) + rw[PPW:2 * PPW] * (wx1 * wy0)
            + rw[2 * PPW:3 * PPW] * (wx0 * wy1) + rw[3 * PPW:] * (wx1 * wy1))

    nf = nf_scr[...]
    s1 = jnp.sum(nf, axis=0, keepdims=True)
    s2 = jnp.sum(nf * nf, axis=0, keepdims=True)
    mean = s1 * (1.0 / NPTS)
    var = s2 * (1.0 / NPTS) - mean * mean
    inv = lax.rsqrt(var + 1e-5)

    node = lax.broadcasted_iota(jnp.int32, (TILE, 1), 0) % NODE

    def tile_fn(t, carry):
        r0 = pl.multiple_of(t * TILE, TILE)
        xx = (nf_scr[pl.ds(r0, TILE), :] - mean) * inv
        for wgt, bias in ((w1, b1), (w2, b2), (w3, b3), (w4, b4)):
            agg = _ring_avg(xx, node)
            cat = jnp.concatenate([xx, agg], axis=1)
            xx = jnp.maximum(
                jnp.dot(cat, wgt[...], preferred_element_type=jnp.float32)
                + bias[...], 0.0)
        h = jnp.maximum(
            jnp.dot(xx, p1[...], preferred_element_type=jnp.float32) + q1[...], 0.0)
        h = jnp.maximum(
            jnp.dot(h, p2[...], preferred_element_type=jnp.float32) + q2[...], 0.0)
        off = jnp.dot(h, p3[...], preferred_element_type=jnp.float32) + q3[...]
        pts = polys_ref[pl.ds(r0, TILE), :]
        out_ref[pl.ds(r0, TILE), :] = jnp.clip(
            pts + jnp.clip(off, -16.0, 16.0), 0.0, float(W - 1))
        return carry

    lax.fori_loop(0, NPTS // TILE, tile_fn, 0)


_tc_call = pl.pallas_call(
    _tc_body,
    out_shape=jax.ShapeDtypeStruct((NPTS, 2), jnp.float32),
    scratch_shapes=[pltpu.VMEM((NPTS, CPAD), jnp.float32)],
)


def _prep_weights(p):
    (w1, b1), (w2, b2), (w3, b3), (w4, b4) = p["gconvs"]
    w1p = jnp.zeros((2 * CPAD, w1.shape[1]), jnp.float32)
    w1p = w1p.at[0:C_REAL].set(w1[0:C_REAL])
    w1p = w1p.at[CPAD:CPAD + C_REAL].set(w1[C_REAL:2 * C_REAL])
    (p1, q1), (p2, q2), (p3, q3) = p["pred"]
    p1p = jnp.zeros((128, 128), jnp.float32).at[:, :64].set(p1)
    q1p = jnp.zeros((128,), jnp.float32).at[:64].set(q1)
    p2p = jnp.zeros((128, 128), jnp.float32).at[:64, :4].set(p2)
    q2p = jnp.zeros((128,), jnp.float32).at[:4].set(q2)
    p3p = jnp.zeros((128, 2), jnp.float32).at[:4].set(p3)
    return (w1p, b1[None], w2, b2[None], w3, b3[None], w4, b4[None],
            p1p, q1p[None], p2p, q2p[None], p3p, q3[None])


def kernel(embed_feature, ignore_tags, proposal_points, params):
    del ignore_tags
    b, c, h, w = embed_feature.shape
    feat = jnp.transpose(embed_feature, (0, 2, 3, 1))
    feat = jnp.pad(feat, ((0, 0), (0, 0), (0, 0), (0, CPAD - c)))
    table = feat.reshape(b * h * w, CPAD)

    init = proposal_points.reshape(-1, NODE, 2)
    polys = init.reshape(NPTS, 2)
    outs = [init]
    for it in range(len(params)):
        rows = _sc_gather_call()(polys[:, 0], polys[:, 1], table)
        polys = _tc_call(rows, polys, *_prep_weights(params[it]))
        outs.append(polys.reshape(-1, NODE, 2))
    return tuple(outs)


# speedup vs baseline: 2.0436x; 1.0635x over previous
import functools

import jax
import jax.numpy as jnp
from jax import lax
from jax.experimental import pallas as pl
from jax.experimental.pallas import tpu as pltpu
from jax.experimental.pallas import tpu_sc as plsc

NODE = 20
N_INST = 256
NPTS = N_INST * NODE
C_REAL = 36
CPAD = 48
H = W = 256
NW = 32
PPW = NPTS // NW
ROWS_PW = 4 * PPW
IDX_CH = 128
N_CH = ROWS_PW // IDX_CH
TILE = 640
INV128 = 1.0 / 128.0


def _sc_body(px_hbm, py_hbm, table_hbm, out_hbm, px_v, py_v, idx_v, rows_v, sem):
    cid = lax.axis_index("c")
    sid = lax.axis_index("s")
    wid = sid * 2 + cid
    base = pl.multiple_of(wid * PPW, PPW)
    pltpu.sync_copy(px_hbm.at[pl.ds(base, PPW)], px_v)
    pltpu.sync_copy(py_hbm.at[pl.ds(base, PPW)], py_v)
    rowbase = (wid // 8) * (H * W)
    for g in range(PPW // 16):
        pxg = px_v[pl.ds(g * 16, 16)]
        pyg = py_v[pl.ds(g * 16, 16)]
        x = (pxg * INV128 - 1.0 + 1.0) * 127.5
        y = (pyg * INV128 - 1.0 + 1.0) * 127.5
        x0 = jnp.clip(x.astype(jnp.int32), 0, W - 1)
        y0 = jnp.clip(y.astype(jnp.int32), 0, H - 1)
        x1 = jnp.minimum(x0 + 1, W - 1)
        y1 = jnp.minimum(y0 + 1, H - 1)
        r0 = rowbase + y0 * W
        r1 = rowbase + y1 * W
        for k, r in enumerate((r0 + x0, r0 + x1, r1 + x0, r1 + x1)):
            off = k * PPW + g * 16
            idx_v[off // IDX_CH, pl.ds(off % IDX_CH, 16)] = r
    handles = [
        pltpu.async_copy(table_hbm.at[idx_v.at[j]],
                         rows_v.at[pl.ds(j * IDX_CH, IDX_CH)], sem)
        for j in range(N_CH)
    ]
    for h in handles:
        h.wait()
    pltpu.sync_copy(rows_v, out_hbm.at[wid])


@functools.cache
def _sc_gather_call():
    return pl.kernel(
        _sc_body,
        mesh=plsc.VectorSubcoreMesh(core_axis_name="c", subcore_axis_name="s"),
        compiler_params=pltpu.CompilerParams(use_tc_tiling_on_sc=False),
        out_type=jax.ShapeDtypeStruct((NW, ROWS_PW, CPAD), jnp.float32),
        scratch_types=[
            pltpu.VMEM((PPW,), jnp.float32),
            pltpu.VMEM((PPW,), jnp.float32),
            pltpu.VMEM((N_CH, IDX_CH), jnp.int32),
            pltpu.VMEM((ROWS_PW, CPAD), jnp.float32),
            pltpu.SemaphoreType.DMA,
        ],
    )


def _ring_avg(xx, node):
    n = xx.shape[0]
    acc = xx
    for k in (1, 2):
        a = jnp.concatenate([xx[n - k:], xx[:n - k]], axis=0)
        b = jnp.concatenate([xx[NODE - k:], xx[:NODE - k]], axis=0)
        acc = acc + jnp.where(node < k, b, a)
        c = jnp.concatenate([xx[k:], xx[:k]], axis=0)
        d = jnp.concatenate([xx[n - (NODE - k):], xx[:n - (NODE - k)]], axis=0)
        acc = acc + jnp.where(node >= NODE - k, d, c)
    return acc * 0.2


def _tc_body(rows_ref, polys_ref, w1, b1, w2, b2, w3, b3, w4, b4,
             p1, q1, p2, q2, p3, q3, out_ref, nf_scr):
    for wslot in range(NW):
        rw = rows_ref[wslot]
        pts = polys_ref[pl.ds(wslot * PPW, PPW), :]
        x = (pts[:, 0:1] * INV128 - 1.0 + 1.0) * 127.5
        y = (pts[:, 1:2] * INV128 - 1.0 + 1.0) * 127.5
        x0 = jnp.floor(x)
        y0 = jnp.floor(y)
        wx1 = x - x0
        wx0 = 1.0 - wx1
        wy1 = y - y0
        wy0 = 1.0 - wy1
        nf_scr[pl.ds(wslot * PPW, PPW), :] = (
            rw[0:PPW] * (wx0 * wy0) + rw[PPW:2 * PPW] * (wx1 * wy0)
            + rw[2 * PPW:3 * PPW] * (wx0 * wy1) + rw[3 * PPW:] * (wx1 * wy1))

    nf = nf_scr[...]
    s1 = jnp.sum(nf, axis=0, keepdims=True)
    s2 = jnp.sum(nf * nf, axis=0, keepdims=True)
    mean = s1 * (1.0 / NPTS)
    var = s2 * (1.0 / NPTS) - mean * mean
    inv = lax.rsqrt(var + 1e-5)

    node = lax.broadcasted_iota(jnp.int32, (TILE, 1), 0) % NODE

    def tile_fn(t, carry):
        r0 = pl.multiple_of(t * TILE, TILE)
        xx = ((nf_scr[pl.ds(r0, TILE), :] - mean) * inv).astype(jnp.bfloat16)
        for wgt, bias in ((w1, b1), (w2, b2), (w3, b3), (w4, b4)):
            agg = _ring_avg(xx, node)
            cat = jnp.concatenate([xx, agg], axis=1)
            xx = jnp.maximum(
                jnp.dot(cat, wgt[...], preferred_element_type=jnp.float32)
                + bias[...], 0.0).astype(jnp.bfloat16)
        h = jnp.maximum(
            jnp.dot(xx, p1[...], preferred_element_type=jnp.float32)
            + q1[...], 0.0).astype(jnp.bfloat16)
        h = jnp.maximum(
            jnp.dot(h, p2[...], preferred_element_type=jnp.float32)
            + q2[...], 0.0).astype(jnp.bfloat16)
        off = jnp.dot(h, p3[...], preferred_element_type=jnp.float32) + q3[...]
        pts = polys_ref[pl.ds(r0, TILE), :]
        out_ref[pl.ds(r0, TILE), :] = jnp.clip(
            pts + jnp.clip(off, -16.0, 16.0), 0.0, float(W - 1))
        return carry

    lax.fori_loop(0, NPTS // TILE, tile_fn, 0)


_tc_call = pl.pallas_call(
    _tc_body,
    out_shape=jax.ShapeDtypeStruct((NPTS, 2), jnp.float32),
    scratch_shapes=[pltpu.VMEM((NPTS, CPAD), jnp.float32)],
)


def _prep_weights(p):
    (w1, b1), (w2, b2), (w3, b3), (w4, b4) = p["gconvs"]
    w1p = jnp.zeros((2 * CPAD, w1.shape[1]), jnp.float32)
    w1p = w1p.at[0:C_REAL].set(w1[0:C_REAL])
    w1p = w1p.at[CPAD:CPAD + C_REAL].set(w1[C_REAL:2 * C_REAL])
    (p1, q1), (p2, q2), (p3, q3) = p["pred"]
    p1p = jnp.zeros((128, 128), jnp.float32).at[:, :64].set(p1)
    q1p = jnp.zeros((128,), jnp.float32).at[:64].set(q1)
    p2p = jnp.zeros((128, 128), jnp.float32).at[:64, :4].set(p2)
    q2p = jnp.zeros((128,), jnp.float32).at[:4].set(q2)
    p3p = jnp.zeros((128, 2), jnp.float32).at[:4].set(p3)
    bf = lambda a: a.astype(jnp.bfloat16)
    return (bf(w1p), b1[None], bf(w2), b2[None], bf(w3), b3[None],
            bf(w4), b4[None], bf(p1p), q1p[None], bf(p2p), q2p[None],
            bf(p3p), q3[None])


def kernel(embed_feature, ignore_tags, proposal_points, params):
    del ignore_tags
    b, c, h, w = embed_feature.shape
    feat = jnp.transpose(embed_feature, (0, 2, 3, 1))
    feat = jnp.pad(feat, ((0, 0), (0, 0), (0, 0), (0, CPAD - c)))
    table = feat.reshape(b * h * w, CPAD)

    init = proposal_points.reshape(-1, NODE, 2)
    polys = init.reshape(NPTS, 2)
    outs = [init]
    for it in range(len(params)):
        rows = _sc_gather_call()(polys[:, 0], polys[:, 1], table)
        polys = _tc_call(rows, polys, *_prep_weights(params[it]))
        outs.append(polys.reshape(-1, NODE, 2))
    return tuple(outs)


# speedup vs baseline: 2.2288x; 1.0907x over previous
import functools

import jax
import jax.numpy as jnp
from jax import lax
from jax.experimental import pallas as pl
from jax.experimental.pallas import tpu as pltpu
from jax.experimental.pallas import tpu_sc as plsc

NODE = 20
N_INST = 256
NPTS = N_INST * NODE
C_REAL = 36
CPAD = 64
H = W = 256
NW = 32
PPW = NPTS // NW
ROWS_PW = 4 * PPW
IDX_CH = 128
N_CH = ROWS_PW // IDX_CH
TILE = 640
INV128 = 1.0 / 128.0
TBLK = 8192


def _table_body(feat_ref, out_ref):
    x = feat_ref[0].astype(jnp.bfloat16)
    ident = (lax.broadcasted_iota(jnp.int32, (C_REAL, CPAD), 0)
             == lax.broadcasted_iota(jnp.int32, (C_REAL, CPAD), 1)
             ).astype(jnp.bfloat16)
    out_ref[...] = lax.dot_general(
        x, ident, (((0,), (0,)), ((), ())),
        preferred_element_type=jnp.float32).astype(jnp.bfloat16)


_table_call = pl.pallas_call(
    _table_body,
    grid=(4 * (H * W) // TBLK,),
    in_specs=[pl.BlockSpec((1, C_REAL, TBLK),
                           lambda i: (i // (H * W // TBLK), 0, i % (H * W // TBLK)))],
    out_specs=pl.BlockSpec((TBLK, CPAD), lambda i: (i, 0)),
    out_shape=jax.ShapeDtypeStruct((4 * H * W, CPAD), jnp.bfloat16),
)


def _sc_body(px_hbm, py_hbm, table_hbm, out_hbm, px_v, py_v, idx_v, rows_v, sem):
    cid = lax.axis_index("c")
    sid = lax.axis_index("s")
    wid = sid * 2 + cid
    base = pl.multiple_of(wid * PPW, PPW)
    pltpu.sync_copy(px_hbm.at[pl.ds(base, PPW)], px_v)
    pltpu.sync_copy(py_hbm.at[pl.ds(base, PPW)], py_v)
    rowbase = (wid // 8) * (H * W)
    for g in range(PPW // 16):
        pxg = px_v[pl.ds(g * 16, 16)]
        pyg = py_v[pl.ds(g * 16, 16)]
        x = (pxg * INV128 - 1.0 + 1.0) * 127.5
        y = (pyg * INV128 - 1.0 + 1.0) * 127.5
        x0 = jnp.clip(x.astype(jnp.int32), 0, W - 1)
        y0 = jnp.clip(y.astype(jnp.int32), 0, H - 1)
        x1 = jnp.minimum(x0 + 1, W - 1)
        y1 = jnp.minimum(y0 + 1, H - 1)
        r0 = rowbase + y0 * W
        r1 = rowbase + y1 * W
        for k, r in enumerate((r0 + x0, r0 + x1, r1 + x0, r1 + x1)):
            off = k * PPW + g * 16
            idx_v[off // IDX_CH, pl.ds(off % IDX_CH, 16)] = r
    handles = [
        pltpu.async_copy(table_hbm.at[idx_v.at[j]],
                         rows_v.at[pl.ds(j * IDX_CH, IDX_CH)], sem)
        for j in range(N_CH)
    ]
    for h in handles:
        h.wait()
    for k in range(4):
        pltpu.sync_copy(rows_v.at[pl.ds(k * PPW, PPW)],
                        out_hbm.at[k, pl.ds(base, PPW)])


@functools.cache
def _sc_gather_call():
    return pl.kernel(
        _sc_body,
        mesh=plsc.VectorSubcoreMesh(core_axis_name="c", subcore_axis_name="s"),
        compiler_params=pltpu.CompilerParams(use_tc_tiling_on_sc=False),
        out_type=jax.ShapeDtypeStruct((4, NPTS, CPAD), jnp.bfloat16),
        scratch_types=[
            pltpu.VMEM((PPW,), jnp.float32),
            pltpu.VMEM((PPW,), jnp.float32),
            pltpu.VMEM((N_CH, IDX_CH), jnp.int32),
            pltpu.VMEM((ROWS_PW, CPAD), jnp.bfloat16),
            pltpu.SemaphoreType.DMA,
        ],
    )


def _ring_avg(xx, node):
    n = xx.shape[0]
    acc = xx
    for k in (1, 2):
        a = jnp.concatenate([xx[n - k:], xx[:n - k]], axis=0)
        b = jnp.concatenate([xx[NODE - k:], xx[:NODE - k]], axis=0)
        acc = acc + jnp.where(node < k, b, a)
        c = jnp.concatenate([xx[k:], xx[:k]], axis=0)
        d = jnp.concatenate([xx[n - (NODE - k):], xx[:n - (NODE - k)]], axis=0)
        acc = acc + jnp.where(node >= NODE - k, d, c)
    return acc * 0.2


def _tc_body(rows_ref, px_ref, py_ref, w1, b1, w2, b2, w3, b3, w4, b4,
             p1, q1, p2, q2, p3, q3, out_ref, pxo_ref, pyo_ref, nf_scr):
    px = px_ref[...][:, None]
    py = py_ref[...][:, None]
    x = (px * INV128 - 1.0 + 1.0) * 127.5
    y = (py * INV128 - 1.0 + 1.0) * 127.5
    wx1 = x - jnp.floor(x)
    wx0 = 1.0 - wx1
    wy1 = y - jnp.floor(y)
    wy0 = 1.0 - wy1
    nf_scr[...] = (rows_ref[0].astype(jnp.float32) * (wx0 * wy0)
                   + rows_ref[1].astype(jnp.float32) * (wx1 * wy0)
                   + rows_ref[2].astype(jnp.float32) * (wx0 * wy1)
                   + rows_ref[3].astype(jnp.float32) * (wx1 * wy1))

    nf = nf_scr[...]
    s1 = jnp.sum(nf, axis=0, keepdims=True)
    s2 = jnp.sum(nf * nf, axis=0, keepdims=True)
    mean = s1 * (1.0 / NPTS)
    var = s2 * (1.0 / NPTS) - mean * mean
    inv = lax.rsqrt(var + 1e-5)

    node = lax.broadcasted_iota(jnp.int32, (TILE, 1), 0) % NODE

    def tile_fn(t, carry):
        r0 = pl.multiple_of(t * TILE, TILE)
        xx = ((nf_scr[pl.ds(r0, TILE), :] - mean) * inv).astype(jnp.bfloat16)
        for wgt, bias in ((w1, b1), (w2, b2), (w3, b3), (w4, b4)):
            agg = _ring_avg(xx, node)
            cat = jnp.concatenate([xx, agg], axis=1)
            xx = jnp.maximum(
                jnp.dot(cat, wgt[...], preferred_element_type=jnp.float32)
                + bias[...], 0.0).astype(jnp.bfloat16)
        h = jnp.maximum(
            jnp.dot(xx, p1[...], preferred_element_type=jnp.float32)
            + q1[...], 0.0).astype(jnp.bfloat16)
        h = jnp.maximum(
            jnp.dot(h, p2[...], preferred_element_type=jnp.float32)
            + q2[...], 0.0).astype(jnp.bfloat16)
        off = jnp.dot(h, p3[...], preferred_element_type=jnp.float32) + q3[...]
        pts = jnp.concatenate(
            [px_ref[pl.ds(r0, TILE)][:, None], py_ref[pl.ds(r0, TILE)][:, None]],
            axis=1)
        newp = jnp.clip(pts + jnp.clip(off, -16.0, 16.0), 0.0, float(W - 1))
        out_ref[pl.ds(r0, TILE), :] = newp
        pxo_ref[pl.ds(r0, TILE)] = newp[:, 0]
        pyo_ref[pl.ds(r0, TILE)] = newp[:, 1]
        return carry

    lax.fori_loop(0, NPTS // TILE, tile_fn, 0)


_tc_call = pl.pallas_call(
    _tc_body,
    out_shape=(jax.ShapeDtypeStruct((NPTS, 2), jnp.float32),
               jax.ShapeDtypeStruct((NPTS,), jnp.float32),
               jax.ShapeDtypeStruct((NPTS,), jnp.float32)),
    scratch_shapes=[pltpu.VMEM((NPTS, CPAD), jnp.float32)],
)


def _prep_weights(p):
    (w1, b1), (w2, b2), (w3, b3), (w4, b4) = p["gconvs"]
    w1p = jnp.zeros((2 * CPAD, w1.shape[1]), jnp.float32)
    w1p = w1p.at[0:C_REAL].set(w1[0:C_REAL])
    w1p = w1p.at[CPAD:CPAD + C_REAL].set(w1[C_REAL:2 * C_REAL])
    (p1, q1), (p2, q2), (p3, q3) = p["pred"]
    p1p = jnp.zeros((128, 128), jnp.float32).at[:, :64].set(p1)
    q1p = jnp.zeros((128,), jnp.float32).at[:64].set(q1)
    p2p = jnp.zeros((128, 128), jnp.float32).at[:64, :4].set(p2)
    q2p = jnp.zeros((128,), jnp.float32).at[:4].set(q2)
    p3p = jnp.zeros((128, 2), jnp.float32).at[:4].set(p3)
    bf = lambda a: a.astype(jnp.bfloat16)
    return (bf(w1p), b1[None], bf(w2), b2[None], bf(w3), b3[None],
            bf(w4), b4[None], bf(p1p), q1p[None], bf(p2p), q2p[None],
            bf(p3p), q3[None])


def kernel(embed_feature, ignore_tags, proposal_points, params):
    del ignore_tags
    b, c, h, w = embed_feature.shape
    table = _table_call(embed_feature.reshape(b, c, h * w))

    init = proposal_points.reshape(-1, NODE, 2)
    polys = init.reshape(NPTS, 2)
    px, py = polys[:, 0], polys[:, 1]
    outs = [init]
    for it in range(len(params)):
        rows = _sc_gather_call()(px, py, table)
        polys, px, py = _tc_call(rows, px, py, *_prep_weights(params[it]))
        outs.append(polys.reshape(-1, NODE, 2))
    return tuple(outs)
